# Initial kernel scaffold; baseline (speedup 1.0000x reference)
import jax
import jax.numpy as jnp
from jax.experimental import pallas as pl


def kernel(x, edge_index, batch, W1_0, b1_0, W2_0, b2_0, W1_1, b1_1, W2_1, b2_1, W1_2, b1_2, W2_2, b2_2, Wp1, bp1, Wp2, bp2):
    raise NotImplementedError("write your pallas kernel here")


# speedup vs baseline: 2.6134x; 2.6134x over previous
import functools

import jax
import jax.numpy as jnp
from jax import lax
from jax.experimental import pallas as pl
from jax.experimental.pallas import tpu as pltpu
from jax.experimental.pallas import tpu_sc as plsc

_N = 10000
_E = 320000
_D = 128
_G = 64
_NC = 2
_NS = 16
_NW = _NC * _NS
_CHUNK = 128
_CHUNKS = 80
_EPT = _CHUNKS * _CHUNK
_EPAD = _NW * _EPT
_ACC_N = 10240
_RPT = _ACC_N // _NS
_DUMMY = _N

_ROWB = 1000
_NBLK = _N // _ROWB


def _agg_body(h_hbm, src_hbm, dst_hbm, out_hbm,
              src_v, dst_v, rows_v, acc_sh, sem):
    c = lax.axis_index("c")
    s = lax.axis_index("s")
    wid = s * _NC + c

    pltpu.sync_copy(src_hbm.at[wid], src_v)
    pltpu.sync_copy(dst_hbm.at[wid], dst_v)

    zeros16 = jnp.zeros((16,), jnp.float32)

    def _zrow(i, carry):
        for k in range(_D // 16):
            rows_v[i, pl.ds(k * 16, 16)] = zeros16
        return carry

    lax.fori_loop(0, _CHUNK, _zrow, 0)
    base = s * _RPT
    for k in range(_RPT // _CHUNK):
        pltpu.sync_copy(rows_v, acc_sh.at[pl.ds(base + k * _CHUNK, _CHUNK)])
    plsc.subcore_barrier()

    def _chunk(j, carry):
        pltpu.async_copy(h_hbm.at[src_v.at[j]], rows_v, sem).wait()
        pltpu.sync_copy(rows_v, acc_sh.at[dst_v.at[j]], add=True)
        return carry

    lax.fori_loop(0, _CHUNKS, _chunk, 0)
    plsc.subcore_barrier()

    pltpu.sync_copy(acc_sh.at[pl.ds(base, _RPT)],
                    out_hbm.at[c, pl.ds(base, _RPT)])


@jax.jit
def _sc_aggregate(h, src_p, dst_p):
    mesh = plsc.VectorSubcoreMesh(core_axis_name="c", subcore_axis_name="s")
    run = pl.kernel(
        _agg_body,
        out_type=jax.ShapeDtypeStruct((_NC, _ACC_N, _D), jnp.float32),
        mesh=mesh,
        scratch_types=[
            pltpu.VMEM((_CHUNKS, _CHUNK), jnp.int32),
            pltpu.VMEM((_CHUNKS, _CHUNK), jnp.int32),
            pltpu.VMEM((_CHUNK, _D), jnp.float32),
            pltpu.VMEM_SHARED((_ACC_N, _D), jnp.float32),
            pltpu.SemaphoreType.DMA,
        ],
    )
    return run(h, src_p, dst_p)


def _mlp_body(x_ref, a0_ref, a1_ref, w1_ref, b1_ref, w2_ref, b2_ref, o_ref):
    t = x_ref[...] + a0_ref[...] + a1_ref[...]
    u = jnp.dot(t, w1_ref[...], preferred_element_type=jnp.float32) + b1_ref[...]
    u = jnp.where(u > 0, u, jnp.exp(jnp.minimum(u, 0.0)) - 1.0)
    o_ref[...] = jnp.dot(u, w2_ref[...], preferred_element_type=jnp.float32) + b2_ref[...]


def _tc_mlp(x, a0, a1, w1, b1, w2, b2):
    full = lambda shape: pl.BlockSpec(shape, lambda i: (0, 0))
    rowb = pl.BlockSpec((_ROWB, _D), lambda i: (i, 0))
    return pl.pallas_call(
        _mlp_body,
        grid=(_NBLK,),
        in_specs=[rowb, rowb, rowb,
                  full(w1.shape), full((1, _D)), full(w2.shape), full((1, _D))],
        out_specs=rowb,
        out_shape=jax.ShapeDtypeStruct((_N, _D), jnp.float32),
    )(x, a0, a1, w1, b1.reshape(1, _D), w2, b2.reshape(1, _D))


def _readout_body(h_ref, b_ref, wp1_ref, bp1_ref, wp2_ref, bp2_ref, o_ref,
                  sum_s, max_s):
    i = pl.program_id(0)

    @pl.when(i == 0)
    def _():
        sum_s[...] = jnp.zeros_like(sum_s)
        max_s[...] = jnp.full_like(max_s, -jnp.inf)

    h = h_ref[...]
    b = b_ref[0]
    gid = lax.broadcasted_iota(jnp.int32, (_ROWB, _G), 1)
    oh = (b == gid).astype(jnp.float32)
    sum_s[...] += lax.dot_general(oh, h, (((0,), (0,)), ((), ())),
                                  preferred_element_type=jnp.float32)
    parts = []
    for g in range(_G):
        mg = jnp.max(jnp.where(b == g, h, -jnp.inf), axis=0,
                     keepdims=True)
        parts.append(mg)
    max_s[...] = jnp.maximum(max_s[...], jnp.concatenate(parts, axis=0))

    @pl.when(i == _NBLK - 1)
    def _():
        r = jnp.concatenate([sum_s[...], max_s[...]], axis=1)
        u = jnp.dot(r, wp1_ref[...], preferred_element_type=jnp.float32) \
            + bp1_ref[...]
        u = jnp.where(u > 0, u, jnp.exp(jnp.minimum(u, 0.0)) - 1.0)
        o_ref[...] = jnp.dot(u, wp2_ref[...],
                             preferred_element_type=jnp.float32) + bp2_ref[...]


def _tc_readout(h, batch3, wp1, bp1, wp2, bp2):
    full = lambda shape: pl.BlockSpec(shape, lambda i: tuple(0 for _ in shape))
    return pl.pallas_call(
        _readout_body,
        grid=(_NBLK,),
        in_specs=[
            pl.BlockSpec((_ROWB, _D), lambda i: (i, 0)),
            pl.BlockSpec((1, _ROWB, 1), lambda i: (i, 0, 0)),
            full(wp1.shape), full((1, 128)), full(wp2.shape), full((1, 1)),
        ],
        out_specs=full((_G, 1)),
        out_shape=jax.ShapeDtypeStruct((_G, 1), jnp.float32),
        scratch_shapes=[
            pltpu.VMEM((_G, _D), jnp.float32),
            pltpu.VMEM((_G, _D), jnp.float32),
        ],
    )(h, batch3, wp1, bp1.reshape(1, 128), wp2, bp2.reshape(1, 1))


def kernel(x, edge_index, batch,
           W1_0, b1_0, W2_0, b2_0,
           W1_1, b1_1, W2_1, b2_1,
           W1_2, b1_2, W2_2, b2_2,
           Wp1, bp1, Wp2, bp2):
    pad = _EPAD - _E
    src_p = jnp.concatenate(
        [edge_index[0], jnp.zeros((pad,), jnp.int32)]
    ).reshape(_NW, _CHUNKS, _CHUNK)
    dst_p = jnp.concatenate(
        [edge_index[1], jnp.full((pad,), _DUMMY, jnp.int32)]
    ).reshape(_NW, _CHUNKS, _CHUNK)
    batch3 = batch.reshape(_NBLK, _ROWB, 1)

    h = x
    for (w1, b1, w2, b2) in ((W1_0, b1_0, W2_0, b2_0),
                             (W1_1, b1_1, W2_1, b2_1),
                             (W1_2, b1_2, W2_2, b2_2)):
        acc = _sc_aggregate(h, src_p, dst_p)
        h = _tc_mlp(h, acc[0, :_N], acc[1, :_N], w1, b1, w2, b2)

    return _tc_readout(h, batch3, Wp1, bp1, Wp2, bp2)


# speedup vs baseline: 2.8835x; 1.1034x over previous
import functools

import jax
import jax.numpy as jnp
from jax import lax
from jax.experimental import pallas as pl
from jax.experimental.pallas import tpu as pltpu
from jax.experimental.pallas import tpu_sc as plsc

_N = 10000
_E = 320000
_D = 128
_G = 64
_NC = 2
_NS = 16
_NW = _NC * _NS
_CHUNK = 128
_CHUNKS = 80
_HALF = _CHUNKS // 2
_EPT = _CHUNKS * _CHUNK
_EPAD = _NW * _EPT
_ACC_N = 10112
_RPT = _ACC_N // _NS
_DUMMY = _N

_ROWB = 1000
_NBLK = _N // _ROWB


def _agg_body(h_hbm, src_hbm, dst_hbm, out_hbm,
              src_v, dst_v, rows_a, rows_b, acc_sh, sem_a, sem_b):
    c = lax.axis_index("c")
    s = lax.axis_index("s")
    wid = s * _NC + c

    zeros16 = jnp.zeros((16,), jnp.float32)

    def _zrow(i, carry):
        for k in range(_D // 16):
            rows_a[i, pl.ds(k * 16, 16)] = zeros16
        return carry

    lax.fori_loop(0, _CHUNK, _zrow, 0)
    base = s * _RPT
    for k in range(_RPT // _CHUNK):
        pltpu.sync_copy(rows_a, acc_sh.at[pl.ds(base + k * _CHUNK, _CHUNK)])
    rem = _RPT % _CHUNK
    if rem:
        pltpu.sync_copy(rows_a.at[pl.ds(0, rem)],
                        acc_sh.at[pl.ds(base + (_RPT // _CHUNK) * _CHUNK, rem)])
    plsc.subcore_barrier()

    for half in range(2):
        pltpu.sync_copy(src_hbm.at[wid, pl.ds(half * _HALF, _HALF)], src_v)
        pltpu.sync_copy(dst_hbm.at[wid, pl.ds(half * _HALF, _HALF)], dst_v)
        pltpu.async_copy(h_hbm.at[src_v.at[0]], rows_a, sem_a)

        def _pair(jj, carry):
            j0 = 2 * jj
            pltpu.async_copy(h_hbm.at[src_v.at[j0 + 1]], rows_b, sem_b)
            pltpu.make_async_copy(h_hbm.at[src_v.at[j0]], rows_a, sem_a).wait()
            pltpu.sync_copy(rows_a, acc_sh.at[dst_v.at[j0]], add=True)

            @pl.when(jj < _HALF // 2 - 1)
            def _():
                pltpu.async_copy(h_hbm.at[src_v.at[j0 + 2]], rows_a, sem_a)

            pltpu.make_async_copy(h_hbm.at[src_v.at[j0 + 1]], rows_b, sem_b).wait()
            pltpu.sync_copy(rows_b, acc_sh.at[dst_v.at[j0 + 1]], add=True)
            return carry

        lax.fori_loop(0, _HALF // 2, _pair, 0)
    plsc.subcore_barrier()

    pltpu.sync_copy(acc_sh.at[pl.ds(base, _RPT)],
                    out_hbm.at[c, pl.ds(base, _RPT)])


@jax.jit
def _sc_aggregate(h, src_p, dst_p):
    mesh = plsc.VectorSubcoreMesh(core_axis_name="c", subcore_axis_name="s")
    run = pl.kernel(
        _agg_body,
        out_type=jax.ShapeDtypeStruct((_NC, _ACC_N, _D), jnp.float32),
        mesh=mesh,
        scratch_types=[
            pltpu.VMEM((_HALF, _CHUNK), jnp.int32),
            pltpu.VMEM((_HALF, _CHUNK), jnp.int32),
            pltpu.VMEM((_CHUNK, _D), jnp.float32),
            pltpu.VMEM((_CHUNK, _D), jnp.float32),
            pltpu.VMEM_SHARED((_ACC_N, _D), jnp.float32),
            pltpu.SemaphoreType.DMA,
            pltpu.SemaphoreType.DMA,
        ],
    )
    return run(h, src_p, dst_p)


def _mlp_body(x_ref, a0_ref, a1_ref, w1_ref, b1_ref, w2_ref, b2_ref, o_ref):
    t = x_ref[...] + a0_ref[...] + a1_ref[...]
    u = jnp.dot(t, w1_ref[...], preferred_element_type=jnp.float32) + b1_ref[...]
    u = jnp.where(u > 0, u, jnp.exp(jnp.minimum(u, 0.0)) - 1.0)
    o_ref[...] = jnp.dot(u, w2_ref[...], preferred_element_type=jnp.float32) + b2_ref[...]


def _tc_mlp(x, a0, a1, w1, b1, w2, b2):
    full = lambda shape: pl.BlockSpec(shape, lambda i: (0, 0))
    rowb = pl.BlockSpec((_ROWB, _D), lambda i: (i, 0))
    return pl.pallas_call(
        _mlp_body,
        grid=(_NBLK,),
        in_specs=[rowb, rowb, rowb,
                  full(w1.shape), full((1, _D)), full(w2.shape), full((1, _D))],
        out_specs=rowb,
        out_shape=jax.ShapeDtypeStruct((_N, _D), jnp.float32),
    )(x, a0, a1, w1, b1.reshape(1, _D), w2, b2.reshape(1, _D))


def _readout_body(h_ref, b_ref, wp1_ref, bp1_ref, wp2_ref, bp2_ref, o_ref,
                  sum_s, max_s):
    i = pl.program_id(0)

    @pl.when(i == 0)
    def _():
        sum_s[...] = jnp.zeros_like(sum_s)
        max_s[...] = jnp.full_like(max_s, -jnp.inf)

    h = h_ref[...]
    b = b_ref[0]
    gid = lax.broadcasted_iota(jnp.int32, (_ROWB, _G), 1)
    oh = (b == gid).astype(jnp.float32)
    sum_s[...] += lax.dot_general(oh, h, (((0,), (0,)), ((), ())),
                                  preferred_element_type=jnp.float32)
    parts = []
    for g in range(_G):
        mg = jnp.max(jnp.where(b == g, h, -jnp.inf), axis=0,
                     keepdims=True)
        parts.append(mg)
    max_s[...] = jnp.maximum(max_s[...], jnp.concatenate(parts, axis=0))

    @pl.when(i == _NBLK - 1)
    def _():
        r = jnp.concatenate([sum_s[...], max_s[...]], axis=1)
        u = jnp.dot(r, wp1_ref[...], preferred_element_type=jnp.float32) \
            + bp1_ref[...]
        u = jnp.where(u > 0, u, jnp.exp(jnp.minimum(u, 0.0)) - 1.0)
        o_ref[...] = jnp.dot(u, wp2_ref[...],
                             preferred_element_type=jnp.float32) + bp2_ref[...]


def _tc_readout(h, batch3, wp1, bp1, wp2, bp2):
    full = lambda shape: pl.BlockSpec(shape, lambda i: tuple(0 for _ in shape))
    return pl.pallas_call(
        _readout_body,
        grid=(_NBLK,),
        in_specs=[
            pl.BlockSpec((_ROWB, _D), lambda i: (i, 0)),
            pl.BlockSpec((1, _ROWB, 1), lambda i: (i, 0, 0)),
            full(wp1.shape), full((1, 128)), full(wp2.shape), full((1, 1)),
        ],
        out_specs=full((_G, 1)),
        out_shape=jax.ShapeDtypeStruct((_G, 1), jnp.float32),
        scratch_shapes=[
            pltpu.VMEM((_G, _D), jnp.float32),
            pltpu.VMEM((_G, _D), jnp.float32),
        ],
    )(h, batch3, wp1, bp1.reshape(1, 128), wp2, bp2.reshape(1, 1))


def kernel(x, edge_index, batch,
           W1_0, b1_0, W2_0, b2_0,
           W1_1, b1_1, W2_1, b2_1,
           W1_2, b1_2, W2_2, b2_2,
           Wp1, bp1, Wp2, bp2):
    pad = _EPAD - _E
    src_p = jnp.concatenate(
        [edge_index[0], jnp.zeros((pad,), jnp.int32)]
    ).reshape(_NW, _CHUNKS, _CHUNK)
    dst_p = jnp.concatenate(
        [edge_index[1], jnp.full((pad,), _DUMMY, jnp.int32)]
    ).reshape(_NW, _CHUNKS, _CHUNK)
    batch3 = batch.reshape(_NBLK, _ROWB, 1)

    h = x
    for (w1, b1, w2, b2) in ((W1_0, b1_0, W2_0, b2_0),
                             (W1_1, b1_1, W2_1, b2_1),
                             (W1_2, b1_2, W2_2, b2_2)):
        acc = _sc_aggregate(h, src_p, dst_p)
        h = _tc_mlp(h, acc[0, :_N], acc[1, :_N], w1, b1, w2, b2)

    return _tc_readout(h, batch3, Wp1, bp1, Wp2, bp2)


# speedup vs baseline: 3.4308x; 1.1898x over previous
import functools

import jax
import jax.numpy as jnp
from jax import lax
from jax.experimental import pallas as pl
from jax.experimental.pallas import tpu as pltpu
from jax.experimental.pallas import tpu_sc as plsc

_N = 10000
_E = 320000
_D = 128
_G = 64
_NC = 2
_NS = 16
_NW = _NC * _NS
_CHUNK = 128
_CHUNKS = 80
_HALF = _CHUNKS // 2
_EPT = _CHUNKS * _CHUNK
_EPAD = _NW * _EPT
_ACC_N = 10112
_RPT = _ACC_N // _NS
_DUMMY = _N

_ROWB = 1000
_NBLK = _N // _ROWB


def _agg_body(h_hbm, src_hbm, dst_hbm, out_hbm,
              src_v, dst_v, rows_a, rows_b, acc_sh, sem_a, sem_b):
    c = lax.axis_index("c")
    s = lax.axis_index("s")
    wid = s * _NC + c

    zeros16 = jnp.zeros((16,), jnp.float32)

    def _zrow(i, carry):
        for k in range(_D // 16):
            rows_a[i, pl.ds(k * 16, 16)] = zeros16
        return carry

    lax.fori_loop(0, _CHUNK, _zrow, 0)
    base = s * _RPT
    for k in range(_RPT // _CHUNK):
        pltpu.sync_copy(rows_a, acc_sh.at[pl.ds(base + k * _CHUNK, _CHUNK)])
    rem = _RPT % _CHUNK
    if rem:
        pltpu.sync_copy(rows_a.at[pl.ds(0, rem)],
                        acc_sh.at[pl.ds(base + (_RPT // _CHUNK) * _CHUNK, rem)])
    plsc.subcore_barrier()

    for half in range(2):
        pltpu.sync_copy(src_hbm.at[wid, pl.ds(half * _HALF, _HALF)], src_v)
        pltpu.sync_copy(dst_hbm.at[wid, pl.ds(half * _HALF, _HALF)], dst_v)
        pltpu.async_copy(h_hbm.at[src_v.at[0]], rows_a, sem_a)

        def _pair(jj, carry):
            j0 = 2 * jj
            pltpu.async_copy(h_hbm.at[src_v.at[j0 + 1]], rows_b, sem_b)
            pltpu.make_async_copy(h_hbm.at[src_v.at[j0]], rows_a, sem_a).wait()
            pltpu.sync_copy(rows_a, acc_sh.at[dst_v.at[j0]], add=True)

            @pl.when(jj < _HALF // 2 - 1)
            def _():
                pltpu.async_copy(h_hbm.at[src_v.at[j0 + 2]], rows_a, sem_a)

            pltpu.make_async_copy(h_hbm.at[src_v.at[j0 + 1]], rows_b, sem_b).wait()
            pltpu.sync_copy(rows_b, acc_sh.at[dst_v.at[j0 + 1]], add=True)
            return carry

        lax.fori_loop(0, _HALF // 2, _pair, 0)
    plsc.subcore_barrier()

    pltpu.sync_copy(acc_sh.at[pl.ds(base, _RPT)],
                    out_hbm.at[c, pl.ds(base, _RPT)])


@jax.jit
def _sc_aggregate(h, src_p, dst_p):
    mesh = plsc.VectorSubcoreMesh(core_axis_name="c", subcore_axis_name="s")
    run = pl.kernel(
        _agg_body,
        out_type=jax.ShapeDtypeStruct((_NC, _ACC_N, _D), jnp.float32),
        mesh=mesh,
        scratch_types=[
            pltpu.VMEM((_HALF, _CHUNK), jnp.int32),
            pltpu.VMEM((_HALF, _CHUNK), jnp.int32),
            pltpu.VMEM((_CHUNK, _D), jnp.float32),
            pltpu.VMEM((_CHUNK, _D), jnp.float32),
            pltpu.VMEM_SHARED((_ACC_N, _D), jnp.float32),
            pltpu.SemaphoreType.DMA,
            pltpu.SemaphoreType.DMA,
        ],
    )
    return run(h, src_p, dst_p)


def _mlp_body(x_ref, a0_ref, a1_ref, w1_ref, b1_ref, w2_ref, b2_ref, o_ref):
    t = x_ref[...] + a0_ref[...] + a1_ref[...]
    u = jnp.dot(t, w1_ref[...], preferred_element_type=jnp.float32) + b1_ref[...]
    u = jnp.where(u > 0, u, jnp.exp(jnp.minimum(u, 0.0)) - 1.0)
    o_ref[...] = jnp.dot(u, w2_ref[...], preferred_element_type=jnp.float32) + b2_ref[...]


def _tc_mlp(x, a0, a1, w1, b1, w2, b2):
    full = lambda shape: pl.BlockSpec(shape, lambda i: (0, 0))
    rowb = pl.BlockSpec((_ROWB, _D), lambda i: (i, 0))
    return pl.pallas_call(
        _mlp_body,
        grid=(_NBLK,),
        in_specs=[rowb, rowb, rowb,
                  full(w1.shape), full((1, _D)), full(w2.shape), full((1, _D))],
        out_specs=rowb,
        out_shape=jax.ShapeDtypeStruct((_N, _D), jnp.float32),
    )(x, a0, a1, w1, b1.reshape(1, _D), w2, b2.reshape(1, _D))


def _readout_body(h_ref, b_ref, wp1_ref, bp1_ref, wp2_ref, bp2_ref, o_ref,
                  sum_s, max_s):
    i = pl.program_id(0)

    @pl.when(i == 0)
    def _():
        sum_s[...] = jnp.zeros_like(sum_s)
        max_s[...] = jnp.full_like(max_s, -jnp.inf)

    h = h_ref[...]
    b = b_ref[0]
    gid = lax.broadcasted_iota(jnp.int32, (_ROWB, _G), 1)
    oh = (b == gid).astype(jnp.float32)
    sum_s[...] += lax.dot_general(oh, h, (((0,), (0,)), ((), ())),
                                  preferred_element_type=jnp.float32)
    parts = []
    for g in range(_G):
        mg = jnp.max(jnp.where(b == g, h, -jnp.inf), axis=0,
                     keepdims=True)
        parts.append(mg)
    max_s[...] = jnp.maximum(max_s[...], jnp.concatenate(parts, axis=0))

    @pl.when(i == _NBLK - 1)
    def _():
        r = jnp.concatenate([sum_s[...], max_s[...]], axis=1)
        u = jnp.dot(r, wp1_ref[...], preferred_element_type=jnp.float32) \
            + bp1_ref[...]
        u = jnp.where(u > 0, u, jnp.exp(jnp.minimum(u, 0.0)) - 1.0)
        o_ref[...] = jnp.dot(u, wp2_ref[...],
                             preferred_element_type=jnp.float32) + bp2_ref[...]


def _tc_readout(h, batch3, wp1, bp1, wp2, bp2):
    full = lambda shape: pl.BlockSpec(shape, lambda i: tuple(0 for _ in shape))
    return pl.pallas_call(
        _readout_body,
        grid=(_NBLK,),
        in_specs=[
            pl.BlockSpec((_ROWB, _D), lambda i: (i, 0)),
            pl.BlockSpec((1, _ROWB, 1), lambda i: (i, 0, 0)),
            full(wp1.shape), full((1, 128)), full(wp2.shape), full((1, 1)),
        ],
        out_specs=full((_G, 1)),
        out_shape=jax.ShapeDtypeStruct((_G, 1), jnp.float32),
        scratch_shapes=[
            pltpu.VMEM((_G, _D), jnp.float32),
            pltpu.VMEM((_G, _D), jnp.float32),
        ],
    )(h, batch3, wp1, bp1.reshape(1, 128), wp2, bp2.reshape(1, 1))


def kernel(x, edge_index, batch,
           W1_0, b1_0, W2_0, b2_0,
           W1_1, b1_1, W2_1, b2_1,
           W1_2, b1_2, W2_2, b2_2,
           Wp1, bp1, Wp2, bp2):
    pad = _EPAD - _E
    dummy_dst = _N + (jnp.arange(pad, dtype=jnp.int32) % (_ACC_N - _N))
    src_p = jnp.concatenate(
        [edge_index[0], jnp.zeros((pad,), jnp.int32)]
    ).reshape(_CHUNKS, _NW, _CHUNK).transpose(1, 0, 2)
    dst_p = jnp.concatenate(
        [edge_index[1], dummy_dst]
    ).reshape(_CHUNKS, _NW, _CHUNK).transpose(1, 0, 2)
    batch3 = batch.reshape(_NBLK, _ROWB, 1)

    h = x
    for (w1, b1, w2, b2) in ((W1_0, b1_0, W2_0, b2_0),
                             (W1_1, b1_1, W2_1, b2_1),
                             (W1_2, b1_2, W2_2, b2_2)):
        acc = _sc_aggregate(h, src_p, dst_p)
        h = _tc_mlp(h, acc[0, :_N], acc[1, :_N], w1, b1, w2, b2)

    return _tc_readout(h, batch3, Wp1, bp1, Wp2, bp2)


# speedup vs baseline: 9.1958x; 2.6804x over previous
import functools

import jax
import jax.numpy as jnp
from jax import lax
from jax.experimental import pallas as pl
from jax.experimental.pallas import tpu as pltpu
from jax.experimental.pallas import tpu_sc as plsc

_N = 10000
_E = 320000
_D = 128
_G = 64
_NC = 2
_NS = 16
_NW = _NC * _NS
_CHUNK = 125
_CHUNKS = 80
_HALF = _CHUNKS // 2
_EPT = _CHUNKS * _CHUNK
_ACC_N = 10112
_RPT = _ACC_N // _NS
_ZB = 120

_ROWB = 1000
_NBLK = _N // _ROWB


def _agg_body(h_hbm, src_hbm, dst_hbm, out_hbm,
              src_v, dst_v, rows_a, rows_b, acc_sh, sem_a, sem_b):
    c = lax.axis_index("c")
    s = lax.axis_index("s")
    wid = s * _NC + c

    zeros16 = jnp.zeros((16,), jnp.float32)

    def _zrow(i, carry):
        for k in range(_D // 16):
            rows_a[i, pl.ds(k * 16, 16)] = zeros16
        return carry

    lax.fori_loop(0, _ZB, _zrow, 0)
    base = s * _RPT
    for k in range(_RPT // _ZB):
        pltpu.sync_copy(rows_a.at[pl.ds(0, _ZB)],
                        acc_sh.at[pl.ds(base + k * _ZB, _ZB)])
    rem = _RPT % _ZB
    if rem:
        pltpu.sync_copy(rows_a.at[pl.ds(0, rem)],
                        acc_sh.at[pl.ds(base + (_RPT // _ZB) * _ZB, rem)])
    plsc.subcore_barrier()

    for half in range(2):
        pltpu.sync_copy(src_hbm.at[wid, pl.ds(half * _HALF, _HALF)], src_v)
        pltpu.sync_copy(dst_hbm.at[wid, pl.ds(half * _HALF, _HALF)], dst_v)
        pltpu.async_copy(h_hbm.at[src_v.at[0]], rows_a, sem_a)

        def _pair(jj, carry):
            j0 = 2 * jj
            pltpu.async_copy(h_hbm.at[src_v.at[j0 + 1]], rows_b, sem_b)
            pltpu.make_async_copy(h_hbm.at[src_v.at[j0]], rows_a, sem_a).wait()
            pltpu.sync_copy(rows_a, acc_sh.at[dst_v.at[j0]], add=True)

            @pl.when(jj < _HALF // 2 - 1)
            def _():
                pltpu.async_copy(h_hbm.at[src_v.at[j0 + 2]], rows_a, sem_a)

            pltpu.make_async_copy(h_hbm.at[src_v.at[j0 + 1]], rows_b, sem_b).wait()
            pltpu.sync_copy(rows_b, acc_sh.at[dst_v.at[j0 + 1]], add=True)
            return carry

        lax.fori_loop(0, _HALF // 2, _pair, 0)
    plsc.subcore_barrier()

    pltpu.sync_copy(acc_sh.at[pl.ds(base, _RPT)],
                    out_hbm.at[c, pl.ds(base, _RPT)])


@jax.jit
def _sc_aggregate(h, src_p, dst_p):
    mesh = plsc.VectorSubcoreMesh(core_axis_name="c", subcore_axis_name="s")
    run = pl.kernel(
        _agg_body,
        out_type=jax.ShapeDtypeStruct((_NC, _ACC_N, _D), jnp.float32),
        mesh=mesh,
        scratch_types=[
            pltpu.VMEM((_HALF, _CHUNK), jnp.int32),
            pltpu.VMEM((_HALF, _CHUNK), jnp.int32),
            pltpu.VMEM((_CHUNK, _D), jnp.float32),
            pltpu.VMEM((_CHUNK, _D), jnp.float32),
            pltpu.VMEM_SHARED((_ACC_N, _D), jnp.float32),
            pltpu.SemaphoreType.DMA,
            pltpu.SemaphoreType.DMA,
        ],
    )
    return run(h, src_p, dst_p)


def _mlp_body(x_ref, a0_ref, a1_ref, w1_ref, b1_ref, w2_ref, b2_ref, o_ref):
    t = x_ref[...] + a0_ref[...] + a1_ref[...]
    u = jnp.dot(t, w1_ref[...], preferred_element_type=jnp.float32) + b1_ref[...]
    u = jnp.where(u > 0, u, jnp.exp(jnp.minimum(u, 0.0)) - 1.0)
    o_ref[...] = jnp.dot(u, w2_ref[...], preferred_element_type=jnp.float32) + b2_ref[...]


def _tc_mlp(x, a0, a1, w1, b1, w2, b2):
    full = lambda shape: pl.BlockSpec(shape, lambda i: (0, 0))
    rowb = pl.BlockSpec((_ROWB, _D), lambda i: (i, 0))
    return pl.pallas_call(
        _mlp_body,
        grid=(_NBLK,),
        in_specs=[rowb, rowb, rowb,
                  full(w1.shape), full((1, _D)), full(w2.shape), full((1, _D))],
        out_specs=rowb,
        out_shape=jax.ShapeDtypeStruct((_N, _D), jnp.float32),
    )(x, a0, a1, w1, b1.reshape(1, _D), w2, b2.reshape(1, _D))


def _readout_body(h_ref, b_ref, wp1_ref, bp1_ref, wp2_ref, bp2_ref, o_ref,
                  sum_s, max_s):
    i = pl.program_id(0)

    @pl.when(i == 0)
    def _():
        sum_s[...] = jnp.zeros_like(sum_s)
        max_s[...] = jnp.full_like(max_s, -jnp.inf)

    h = h_ref[...]
    b = b_ref[0]
    gid = lax.broadcasted_iota(jnp.int32, (_ROWB, _G), 1)
    oh = (b == gid).astype(jnp.float32)
    sum_s[...] += lax.dot_general(oh, h, (((0,), (0,)), ((), ())),
                                  preferred_element_type=jnp.float32)
    parts = []
    for g in range(_G):
        mg = jnp.max(jnp.where(b == g, h, -jnp.inf), axis=0,
                     keepdims=True)
        parts.append(mg)
    max_s[...] = jnp.maximum(max_s[...], jnp.concatenate(parts, axis=0))

    @pl.when(i == _NBLK - 1)
    def _():
        r = jnp.concatenate([sum_s[...], max_s[...]], axis=1)
        u = jnp.dot(r, wp1_ref[...], preferred_element_type=jnp.float32) \
            + bp1_ref[...]
        u = jnp.where(u > 0, u, jnp.exp(jnp.minimum(u, 0.0)) - 1.0)
        o_ref[...] = jnp.dot(u, wp2_ref[...],
                             preferred_element_type=jnp.float32) + bp2_ref[...]


def _tc_readout(h, batch3, wp1, bp1, wp2, bp2):
    full = lambda shape: pl.BlockSpec(shape, lambda i: tuple(0 for _ in shape))
    return pl.pallas_call(
        _readout_body,
        grid=(_NBLK,),
        in_specs=[
            pl.BlockSpec((_ROWB, _D), lambda i: (i, 0)),
            pl.BlockSpec((1, _ROWB, 1), lambda i: (i, 0, 0)),
            full(wp1.shape), full((1, 128)), full(wp2.shape), full((1, 1)),
        ],
        out_specs=full((_G, 1)),
        out_shape=jax.ShapeDtypeStruct((_G, 1), jnp.float32),
        scratch_shapes=[
            pltpu.VMEM((_G, _D), jnp.float32),
            pltpu.VMEM((_G, _D), jnp.float32),
        ],
    )(h, batch3, wp1, bp1.reshape(1, 128), wp2, bp2.reshape(1, 1))


def kernel(x, edge_index, batch,
           W1_0, b1_0, W2_0, b2_0,
           W1_1, b1_1, W2_1, b2_1,
           W1_2, b1_2, W2_2, b2_2,
           Wp1, bp1, Wp2, bp2):
    src_p = edge_index[0].reshape(_NW, _CHUNKS, _CHUNK)
    dst_p = edge_index[1].reshape(_NW, _CHUNKS, _CHUNK)
    batch3 = batch.reshape(_NBLK, _ROWB, 1)

    h = x
    for (w1, b1, w2, b2) in ((W1_0, b1_0, W2_0, b2_0),
                             (W1_1, b1_1, W2_1, b2_1),
                             (W1_2, b1_2, W2_2, b2_2)):
        acc = _sc_aggregate(h, src_p, dst_p)
        h = _tc_mlp(h, acc[0, :_N], acc[1, :_N], w1, b1, w2, b2)

    return _tc_readout(h, batch3, Wp1, bp1, Wp2, bp2)


# speedup vs baseline: 11.4000x; 1.2397x over previous
import functools

import jax
import jax.numpy as jnp
from jax import lax
from jax.experimental import pallas as pl
from jax.experimental.pallas import tpu as pltpu
from jax.experimental.pallas import tpu_sc as plsc

_N = 10000
_E = 320000
_D = 128
_G = 64
_NC = 2
_NS = 16
_NW = _NC * _NS
_CHUNK = 125
_CHUNKS = 80
_HALF = _CHUNKS // 2
_EPT = _CHUNKS * _CHUNK
_ACC_N = 10112
_RPT = _ACC_N // _NS
_ZB = 120

_ROWB = 1000
_NBLK = _N // _ROWB


def _agg_body(h_hbm, eidx_hbm, out_hbm,
              src_v, dst_v, rows_a, rows_b, acc_sh, sem_a, sem_b):
    c = lax.axis_index("c")
    s = lax.axis_index("s")
    wid = s * _NC + c

    zeros16 = jnp.zeros((16,), jnp.float32)

    def _zrow(i, carry):
        for k in range(_D // 16):
            rows_a[i, pl.ds(k * 16, 16)] = zeros16
        return carry

    lax.fori_loop(0, _ZB, _zrow, 0)
    base = s * _RPT
    for k in range(_RPT // _ZB):
        pltpu.sync_copy(rows_a.at[pl.ds(0, _ZB)],
                        acc_sh.at[pl.ds(base + k * _ZB, _ZB)])
    rem = _RPT % _ZB
    if rem:
        pltpu.sync_copy(rows_a.at[pl.ds(0, rem)],
                        acc_sh.at[pl.ds(base + (_RPT // _ZB) * _ZB, rem)])
    plsc.subcore_barrier()

    for half in range(2):
        pltpu.sync_copy(eidx_hbm.at[0, wid, pl.ds(half * _HALF, _HALF)], src_v)
        pltpu.sync_copy(eidx_hbm.at[1, wid, pl.ds(half * _HALF, _HALF)], dst_v)
        pltpu.async_copy(h_hbm.at[src_v.at[0]], rows_a, sem_a)

        def _pair(jj, carry):
            j0 = 2 * jj
            pltpu.async_copy(h_hbm.at[src_v.at[j0 + 1]], rows_b, sem_b)
            pltpu.make_async_copy(h_hbm.at[src_v.at[j0]], rows_a, sem_a).wait()
            pltpu.sync_copy(rows_a, acc_sh.at[dst_v.at[j0]], add=True)

            @pl.when(jj < _HALF // 2 - 1)
            def _():
                pltpu.async_copy(h_hbm.at[src_v.at[j0 + 2]], rows_a, sem_a)

            pltpu.make_async_copy(h_hbm.at[src_v.at[j0 + 1]], rows_b, sem_b).wait()
            pltpu.sync_copy(rows_b, acc_sh.at[dst_v.at[j0 + 1]], add=True)
            return carry

        lax.fori_loop(0, _HALF // 2, _pair, 0)
    plsc.subcore_barrier()

    pltpu.sync_copy(acc_sh.at[pl.ds(base, _RPT)],
                    out_hbm.at[c, pl.ds(base, _RPT)])


@jax.jit
def _sc_aggregate(h, eidx):
    mesh = plsc.VectorSubcoreMesh(core_axis_name="c", subcore_axis_name="s")
    run = pl.kernel(
        _agg_body,
        out_type=jax.ShapeDtypeStruct((_NC, _ACC_N, _D), jnp.float32),
        mesh=mesh,
        scratch_types=[
            pltpu.VMEM((_HALF, _CHUNK), jnp.int32),
            pltpu.VMEM((_HALF, _CHUNK), jnp.int32),
            pltpu.VMEM((_CHUNK, _D), jnp.float32),
            pltpu.VMEM((_CHUNK, _D), jnp.float32),
            pltpu.VMEM_SHARED((_ACC_N, _D), jnp.float32),
            pltpu.SemaphoreType.DMA,
            pltpu.SemaphoreType.DMA,
        ],
    )
    return run(h, eidx)


def _mlp_body(x_ref, a0_ref, a1_ref, w1_ref, b1_ref, w2_ref, b2_ref, o_ref):
    t = x_ref[...] + a0_ref[0] + a1_ref[0]
    u = jnp.dot(t, w1_ref[...], preferred_element_type=jnp.float32) + b1_ref[...]
    u = jnp.where(u > 0, u, jnp.exp(jnp.minimum(u, 0.0)) - 1.0)
    o_ref[...] = jnp.dot(u, w2_ref[...], preferred_element_type=jnp.float32) + b2_ref[...]


def _tc_mlp(x, acc, w1, b1, w2, b2):
    full = lambda shape: pl.BlockSpec(shape, lambda i: (0, 0))
    rowb = pl.BlockSpec((_ROWB, _D), lambda i: (i, 0))
    accb = lambda c: pl.BlockSpec((1, _ROWB, _D), lambda i, c=c: (c, i, 0))
    return pl.pallas_call(
        _mlp_body,
        grid=(_NBLK,),
        in_specs=[rowb, accb(0), accb(1),
                  full(w1.shape), full((1, _D)), full(w2.shape), full((1, _D))],
        out_specs=rowb,
        out_shape=jax.ShapeDtypeStruct((_N, _D), jnp.float32),
    )(x, acc, acc, w1, b1.reshape(1, _D), w2, b2.reshape(1, _D))


def _readout_body(h_ref, b_ref, wp1_ref, bp1_ref, wp2_ref, bp2_ref, o_ref,
                  sum_s, max_s):
    i = pl.program_id(0)

    @pl.when(i == 0)
    def _():
        sum_s[...] = jnp.zeros_like(sum_s)
        max_s[...] = jnp.full_like(max_s, -jnp.inf)

    h = h_ref[...]
    b = b_ref[0]
    gid = lax.broadcasted_iota(jnp.int32, (_ROWB, _G), 1)
    oh = (b == gid).astype(jnp.float32)
    sum_s[...] += lax.dot_general(oh, h, (((0,), (0,)), ((), ())),
                                  preferred_element_type=jnp.float32)

    g0 = jnp.min(b)
    nloc = jnp.max(b) - g0 + 1

    def _one_group(l, carry):
        g = g0 + l
        mg = jnp.max(jnp.where(b == g, h, -jnp.inf), axis=0, keepdims=True)
        max_s[pl.ds(g, 1), :] = jnp.maximum(max_s[pl.ds(g, 1), :], mg)
        return carry

    lax.fori_loop(0, nloc, _one_group, 0)

    @pl.when(i == _NBLK - 1)
    def _():
        r = jnp.concatenate([sum_s[...], max_s[...]], axis=1)
        u = jnp.dot(r, wp1_ref[...], preferred_element_type=jnp.float32) \
            + bp1_ref[...]
        u = jnp.where(u > 0, u, jnp.exp(jnp.minimum(u, 0.0)) - 1.0)
        o_ref[...] = jnp.dot(u, wp2_ref[...],
                             preferred_element_type=jnp.float32) + bp2_ref[...]


def _tc_readout(h, batch3, wp1, bp1, wp2, bp2):
    full = lambda shape: pl.BlockSpec(shape, lambda i: tuple(0 for _ in shape))
    return pl.pallas_call(
        _readout_body,
        grid=(_NBLK,),
        in_specs=[
            pl.BlockSpec((_ROWB, _D), lambda i: (i, 0)),
            pl.BlockSpec((1, _ROWB, 1), lambda i: (i, 0, 0)),
            full(wp1.shape), full((1, 128)), full(wp2.shape), full((1, 1)),
        ],
        out_specs=full((_G, 1)),
        out_shape=jax.ShapeDtypeStruct((_G, 1), jnp.float32),
        scratch_shapes=[
            pltpu.VMEM((_G, _D), jnp.float32),
            pltpu.VMEM((_G, _D), jnp.float32),
        ],
    )(h, batch3, wp1, bp1.reshape(1, 128), wp2, bp2.reshape(1, 1))


def kernel(x, edge_index, batch,
           W1_0, b1_0, W2_0, b2_0,
           W1_1, b1_1, W2_1, b2_1,
           W1_2, b1_2, W2_2, b2_2,
           Wp1, bp1, Wp2, bp2):
    eidx = edge_index.reshape(2, _NW, _CHUNKS, _CHUNK)
    batch3 = batch.reshape(_NBLK, _ROWB, 1)

    h = x
    for (w1, b1, w2, b2) in ((W1_0, b1_0, W2_0, b2_0),
                             (W1_1, b1_1, W2_1, b2_1),
                             (W1_2, b1_2, W2_2, b2_2)):
        acc = _sc_aggregate(h, eidx)
        h = _tc_mlp(h, acc, w1, b1, w2, b2)

    return _tc_readout(h, batch3, Wp1, bp1, Wp2, bp2)


# speedup vs baseline: 11.7455x; 1.0303x over previous
import functools

import jax
import jax.numpy as jnp
from jax import lax
from jax.experimental import pallas as pl
from jax.experimental.pallas import tpu as pltpu
from jax.experimental.pallas import tpu_sc as plsc

_N = 10000
_E = 320000
_D = 128
_G = 64
_NC = 2
_NS = 16
_NW = _NC * _NS
_CHUNK = 125
_CHUNKS = 80
_HALF = _CHUNKS // 2
_EPT = _CHUNKS * _CHUNK
_ACC_N = 10112
_RPT = _ACC_N // _NS
_ZB = 120

_ROWB = 1000
_NBLK = _N // _ROWB
_MROWB = 2000
_MNBLK = _N // _MROWB


def _agg_body(h_hbm, eidx_hbm, out_hbm,
              src_v, dst_v, rows_a, rows_b, acc_sh, sem_a, sem_b):
    c = lax.axis_index("c")
    s = lax.axis_index("s")
    wid = s * _NC + c

    zeros16 = jnp.zeros((16,), jnp.float32)

    def _zrow(i, carry):
        for k in range(_D // 16):
            rows_a[i, pl.ds(k * 16, 16)] = zeros16
        return carry

    lax.fori_loop(0, _ZB, _zrow, 0)
    base = s * _RPT
    for k in range(_RPT // _ZB):
        pltpu.sync_copy(rows_a.at[pl.ds(0, _ZB)],
                        acc_sh.at[pl.ds(base + k * _ZB, _ZB)])
    rem = _RPT % _ZB
    if rem:
        pltpu.sync_copy(rows_a.at[pl.ds(0, rem)],
                        acc_sh.at[pl.ds(base + (_RPT // _ZB) * _ZB, rem)])
    plsc.subcore_barrier()

    for half in range(2):
        pltpu.sync_copy(eidx_hbm.at[0, wid, pl.ds(half * _HALF, _HALF)], src_v)
        pltpu.sync_copy(eidx_hbm.at[1, wid, pl.ds(half * _HALF, _HALF)], dst_v)
        pltpu.async_copy(h_hbm.at[src_v.at[0]], rows_a, sem_a)

        def _pair(jj, carry):
            j0 = 2 * jj
            pltpu.async_copy(h_hbm.at[src_v.at[j0 + 1]], rows_b, sem_b)
            pltpu.make_async_copy(h_hbm.at[src_v.at[j0]], rows_a, sem_a).wait()
            pltpu.sync_copy(rows_a, acc_sh.at[dst_v.at[j0]], add=True)

            @pl.when(jj < _HALF // 2 - 1)
            def _():
                pltpu.async_copy(h_hbm.at[src_v.at[j0 + 2]], rows_a, sem_a)

            pltpu.make_async_copy(h_hbm.at[src_v.at[j0 + 1]], rows_b, sem_b).wait()
            pltpu.sync_copy(rows_b, acc_sh.at[dst_v.at[j0 + 1]], add=True)
            return carry

        lax.fori_loop(0, _HALF // 2, _pair, 0)
    plsc.subcore_barrier()

    pltpu.sync_copy(acc_sh.at[pl.ds(base, _RPT)],
                    out_hbm.at[c, pl.ds(base, _RPT)])


@jax.jit
def _sc_aggregate(h, eidx):
    mesh = plsc.VectorSubcoreMesh(core_axis_name="c", subcore_axis_name="s")
    run = pl.kernel(
        _agg_body,
        out_type=jax.ShapeDtypeStruct((_NC, _ACC_N, _D), jnp.float32),
        mesh=mesh,
        scratch_types=[
            pltpu.VMEM((_HALF, _CHUNK), jnp.int32),
            pltpu.VMEM((_HALF, _CHUNK), jnp.int32),
            pltpu.VMEM((_CHUNK, _D), jnp.float32),
            pltpu.VMEM((_CHUNK, _D), jnp.float32),
            pltpu.VMEM_SHARED((_ACC_N, _D), jnp.float32),
            pltpu.SemaphoreType.DMA,
            pltpu.SemaphoreType.DMA,
        ],
    )
    return run(h, eidx)


def _mlp_body(x_ref, a0_ref, a1_ref, w1_ref, b1_ref, w2_ref, b2_ref, o_ref):
    t = x_ref[...] + a0_ref[0] + a1_ref[0]
    u = jnp.dot(t, w1_ref[...], preferred_element_type=jnp.float32) + b1_ref[...]
    u = jnp.where(u > 0, u, jnp.exp(jnp.minimum(u, 0.0)) - 1.0)
    o_ref[...] = jnp.dot(u, w2_ref[...], preferred_element_type=jnp.float32) + b2_ref[...]


def _tc_mlp(x, acc, w1, b1, w2, b2):
    full = lambda shape: pl.BlockSpec(shape, lambda i: (0, 0))
    rowb = pl.BlockSpec((_MROWB, _D), lambda i: (i, 0))
    accb = lambda c: pl.BlockSpec((1, _MROWB, _D), lambda i, c=c: (c, i, 0))
    return pl.pallas_call(
        _mlp_body,
        grid=(_MNBLK,),
        in_specs=[rowb, accb(0), accb(1),
                  full(w1.shape), full((1, _D)), full(w2.shape), full((1, _D))],
        out_specs=rowb,
        out_shape=jax.ShapeDtypeStruct((_N, _D), jnp.float32),
    )(x, acc, acc, w1, b1.reshape(1, _D), w2, b2.reshape(1, _D))


def _readout_body(h_ref, b_ref, wp1_ref, bp1_ref, wp2_ref, bp2_ref, o_ref,
                  sum_s, max_s):
    i = pl.program_id(0)

    @pl.when(i == 0)
    def _():
        sum_s[...] = jnp.zeros_like(sum_s)
        max_s[...] = jnp.full_like(max_s, -jnp.inf)

    h = h_ref[...]
    b = b_ref[0]
    gid = lax.broadcasted_iota(jnp.int32, (_ROWB, _G), 1)
    oh = (b == gid).astype(jnp.float32)
    sum_s[...] += lax.dot_general(oh, h, (((0,), (0,)), ((), ())),
                                  preferred_element_type=jnp.float32)

    g0 = jnp.min(b)
    nloc = jnp.max(b) - g0 + 1

    def _one_group(l, carry):
        g = g0 + l
        mg = jnp.max(jnp.where(b == g, h, -jnp.inf), axis=0, keepdims=True)
        max_s[pl.ds(g, 1), :] = jnp.maximum(max_s[pl.ds(g, 1), :], mg)
        return carry

    lax.fori_loop(0, nloc, _one_group, 0)

    @pl.when(i == _NBLK - 1)
    def _():
        r = jnp.concatenate([sum_s[...], max_s[...]], axis=1)
        u = jnp.dot(r, wp1_ref[...], preferred_element_type=jnp.float32) \
            + bp1_ref[...]
        u = jnp.where(u > 0, u, jnp.exp(jnp.minimum(u, 0.0)) - 1.0)
        o_ref[...] = jnp.dot(u, wp2_ref[...],
                             preferred_element_type=jnp.float32) + bp2_ref[...]


def _tc_readout(h, batch3, wp1, bp1, wp2, bp2):
    full = lambda shape: pl.BlockSpec(shape, lambda i: tuple(0 for _ in shape))
    return pl.pallas_call(
        _readout_body,
        grid=(_NBLK,),
        in_specs=[
            pl.BlockSpec((_ROWB, _D), lambda i: (i, 0)),
            pl.BlockSpec((1, _ROWB, 1), lambda i: (i, 0, 0)),
            full(wp1.shape), full((1, 128)), full(wp2.shape), full((1, 1)),
        ],
        out_specs=full((_G, 1)),
        out_shape=jax.ShapeDtypeStruct((_G, 1), jnp.float32),
        scratch_shapes=[
            pltpu.VMEM((_G, _D), jnp.float32),
            pltpu.VMEM((_G, _D), jnp.float32),
        ],
    )(h, batch3, wp1, bp1.reshape(1, 128), wp2, bp2.reshape(1, 1))


def kernel(x, edge_index, batch,
           W1_0, b1_0, W2_0, b2_0,
           W1_1, b1_1, W2_1, b2_1,
           W1_2, b1_2, W2_2, b2_2,
           Wp1, bp1, Wp2, bp2):
    eidx = edge_index.reshape(2, _NW, _CHUNKS, _CHUNK)
    batch3 = batch.reshape(_NBLK, _ROWB, 1)

    h = x
    for (w1, b1, w2, b2) in ((W1_0, b1_0, W2_0, b2_0),
                             (W1_1, b1_1, W2_1, b2_1),
                             (W1_2, b1_2, W2_2, b2_2)):
        acc = _sc_aggregate(h, eidx)
        h = _tc_mlp(h, acc, w1, b1, w2, b2)

    return _tc_readout(h, batch3, Wp1, bp1, Wp2, bp2)
